# baseline (speedup 1.0000x reference)
import functools

import jax
import jax.numpy as jnp
from jax.experimental import pallas as pl
from jax.experimental.pallas import tpu as pltpu

_VOCAB = 8192
_EMBED = 256
_EPS = 1e-12
_ROWS = 8


_SLAB = 64


def _scan_argmax(c_ref, xb, base, rows, hw):
    nslab = rows // _SLAB

    def step(i, carry):
        acc_v, acc_s = carry
        c = c_ref[pl.ds(base + i * _SLAB, _SLAB), :]
        s = jax.lax.dot_general(
            c, xb, dimension_numbers=(((1,), (0,)), ((), ())),
            preferred_element_type=jnp.float32)
        for r in range(_SLAB // _ROWS):
            srow = s[r * _ROWS:(r + 1) * _ROWS, :]
            pred = srow > acc_v
            acc_v = jnp.maximum(acc_v, srow)
            slab_id = i * (_SLAB // _ROWS) + r
            acc_s = jnp.where(
                pred, jnp.full((_ROWS, hw), 0, jnp.int32) + slab_id, acc_s)
        return acc_v, acc_s

    init = (jnp.full((_ROWS, hw), -jnp.inf, jnp.float32),
            jnp.zeros((_ROWS, hw), jnp.int32))
    acc_v, acc_s = jax.lax.fori_loop(0, nslab, step, init, unroll=2)
    sub = jax.lax.broadcasted_iota(jnp.int32, (_ROWS, hw), 0)
    gidx = acc_s * _ROWS + sub
    mx = jnp.max(acc_v, axis=0)
    idx = jnp.min(jnp.where(acc_v == mx[None, :], gidx, _VOCAB), axis=0)
    return mx, idx


def _vq_body(hw, c_ref, x_ref, o_ref):
    xb = x_ref[0]
    half = _VOCAB // 2
    mx1, i1 = _scan_argmax(c_ref, xb, 0, half, hw)
    mx2, i2 = _scan_argmax(c_ref, xb, half, half, hw)
    i2 = i2 + half
    mx1r = mx1.astype(jnp.bfloat16).astype(jnp.float32)
    o_ref[0, 0] = jnp.where(mx1r >= mx2, i1, i2)


def kernel(x, codebook):
    B, C, H, W = x.shape
    HW = H * W
    cn = codebook / jnp.maximum(
        jnp.linalg.norm(codebook, axis=1, keepdims=True), _EPS)
    cnb = cn.astype(jnp.bfloat16)
    x3 = x.reshape(B, C, HW)
    n = jnp.sqrt(jnp.sum(x3 * x3, axis=1)).reshape(B, 1, HW)
    xt = (x3 / jnp.maximum(n, _EPS)).astype(jnp.bfloat16)
    out = pl.pallas_call(
        functools.partial(_vq_body, HW),
        grid=(B,),
        in_specs=[
            pl.BlockSpec((_VOCAB, _EMBED), lambda i: (0, 0)),
            pl.BlockSpec((1, _EMBED, HW), lambda i: (i, 0, 0)),
        ],
        out_specs=pl.BlockSpec((1, 1, HW), lambda i: (i, 0, 0)),
        out_shape=jax.ShapeDtypeStruct((B, 1, HW), jnp.int32),
    )(cnb, xt)
    return out.reshape(B * HW)


# speedup vs baseline: 3.0943x; 3.0943x over previous
import functools

import jax
import jax.numpy as jnp
from jax.experimental import pallas as pl
from jax.experimental.pallas import tpu as pltpu

_VOCAB = 8192
_EMBED = 256
_EPS = 1e-12
_ROWS = 8


def _scan_argmax(sim, hw):
    rows = sim.shape[0]
    acc_v = sim[0:_ROWS, :]
    acc_s = jnp.zeros((_ROWS, hw), jnp.int32)
    for r in range(1, rows // _ROWS):
        srow = sim[r * _ROWS:(r + 1) * _ROWS, :]
        pred = srow > acc_v
        acc_v = jnp.maximum(acc_v, srow)
        acc_s = jnp.where(pred, jnp.full((_ROWS, hw), r, jnp.int32), acc_s)
    sub = jax.lax.broadcasted_iota(jnp.int32, (_ROWS, hw), 0)
    gidx = acc_s * _ROWS + sub
    mx = jnp.max(acc_v, axis=0)
    idx = jnp.min(jnp.where(acc_v == mx[None, :], gidx, _VOCAB), axis=0)
    return mx, idx


def _vq_body(hw, c_ref, x_ref, o_ref):
    xb = x_ref[0]
    half = _VOCAB // 2
    s1 = jax.lax.dot_general(
        c_ref[:half, :], xb, dimension_numbers=(((1,), (0,)), ((), ())),
        preferred_element_type=jnp.float32)
    s2 = jax.lax.dot_general(
        c_ref[half:, :], xb, dimension_numbers=(((1,), (0,)), ((), ())),
        preferred_element_type=jnp.float32)
    mx1, i1 = _scan_argmax(s1, hw)
    mx2, i2 = _scan_argmax(s2, hw)
    i2 = i2 + half
    mx1r = mx1.astype(jnp.bfloat16).astype(jnp.float32)
    o_ref[0, 0] = jnp.where(mx1r >= mx2, i1, i2)


def kernel(x, codebook):
    B, C, H, W = x.shape
    HW = H * W
    cn = codebook / jnp.maximum(
        jnp.linalg.norm(codebook, axis=1, keepdims=True), _EPS)
    cnb = cn.astype(jnp.bfloat16)
    x3 = x.reshape(B, C, HW)
    n = jnp.sqrt(jnp.sum(x3 * x3, axis=1)).reshape(B, 1, HW)
    xt = (x3 / jnp.maximum(n, _EPS)).astype(jnp.bfloat16)
    out = pl.pallas_call(
        functools.partial(_vq_body, HW),
        grid=(B,),
        in_specs=[
            pl.BlockSpec((_VOCAB, _EMBED), lambda i: (0, 0)),
            pl.BlockSpec((1, _EMBED, HW), lambda i: (i, 0, 0)),
        ],
        out_specs=pl.BlockSpec((1, 1, HW), lambda i: (i, 0, 0)),
        out_shape=jax.ShapeDtypeStruct((B, 1, HW), jnp.int32),
    )(cnb, xt)
    return out.reshape(B * HW)


# speedup vs baseline: 3.1711x; 1.0248x over previous
import functools

import jax
import jax.numpy as jnp
from jax.experimental import pallas as pl
from jax.experimental.pallas import tpu as pltpu

_VOCAB = 8192
_EMBED = 256
_EPS = 1e-12
_ROWS = 8


def _scan_argmax(sim, hw):
    rows = sim.shape[0]
    acc_v = sim[0:_ROWS, :]
    acc_s = jnp.zeros((_ROWS, hw), jnp.int32)
    for r in range(1, rows // _ROWS):
        srow = sim[r * _ROWS:(r + 1) * _ROWS, :]
        pred = srow > acc_v
        acc_v = jnp.maximum(acc_v, srow)
        acc_s = jnp.where(pred, jnp.full((_ROWS, hw), r, jnp.int32), acc_s)
    sub = jax.lax.broadcasted_iota(jnp.int32, (_ROWS, hw), 0)
    gidx = acc_s * _ROWS + sub
    mx = jnp.max(acc_v, axis=0)
    idx = jnp.min(jnp.where(acc_v == mx[None, :], gidx, _VOCAB), axis=0)
    return mx, idx


def _vq_body(hw, nb, c_ref, x_ref, o_ref):
    half = _VOCAB // 2
    for b in range(nb):
        xb = x_ref[b]
        s1 = jax.lax.dot_general(
            c_ref[:half, :], xb, dimension_numbers=(((1,), (0,)), ((), ())),
            preferred_element_type=jnp.float32)
        s2 = jax.lax.dot_general(
            c_ref[half:, :], xb, dimension_numbers=(((1,), (0,)), ((), ())),
            preferred_element_type=jnp.float32)
        mx1, i1 = _scan_argmax(s1, hw)
        mx2, i2 = _scan_argmax(s2, hw)
        i2 = i2 + half
        mx1r = mx1.astype(jnp.bfloat16).astype(jnp.float32)
        o_ref[b, 0] = jnp.where(mx1r >= mx2, i1, i2)


def kernel(x, codebook):
    B, C, H, W = x.shape
    HW = H * W
    cn = codebook / jnp.maximum(
        jnp.linalg.norm(codebook, axis=1, keepdims=True), _EPS)
    cnb = cn.astype(jnp.bfloat16)
    x3 = x.reshape(B, C, HW)
    n = jnp.sqrt(jnp.sum(x3 * x3, axis=1)).reshape(B, 1, HW)
    xt = (x3 / jnp.maximum(n, _EPS)).astype(jnp.bfloat16)
    NB = 2
    out = pl.pallas_call(
        functools.partial(_vq_body, HW, NB),
        grid=(B // NB,),
        in_specs=[
            pl.BlockSpec((_VOCAB, _EMBED), lambda i: (0, 0)),
            pl.BlockSpec((NB, _EMBED, HW), lambda i: (i, 0, 0)),
        ],
        out_specs=pl.BlockSpec((NB, 1, HW), lambda i: (i, 0, 0)),
        out_shape=jax.ShapeDtypeStruct((B, 1, HW), jnp.int32),
    )(cnb, xt)
    return out.reshape(B * HW)
